# Initial kernel scaffold; baseline (speedup 1.0000x reference)
import jax
import jax.numpy as jnp
from jax.experimental import pallas as pl


def kernel(doy, pe):
    raise NotImplementedError("write your pallas kernel here")


# speedup vs baseline: 4.1415x; 4.1415x over previous
import functools
import jax
import jax.numpy as jnp
from jax import lax
from jax.experimental import pallas as pl
from jax.experimental.pallas import tpu as pltpu
from jax.experimental.pallas import tpu_sc as plsc

D = 128
B_ROWS, T_COLS = 4096, 200
B_TOTAL = B_ROWS * T_COLS
NC, NS = 2, 16
NW = NC * NS
B_PER_W = B_TOTAL // NW
CHUNK = 128
N_CHUNKS = B_PER_W // CHUNK


@jax.jit
def _sc_gather(doy_r, pe):
    mesh = plsc.VectorSubcoreMesh(core_axis_name="c", subcore_axis_name="s")

    @functools.partial(
        pl.kernel,
        out_type=jax.ShapeDtypeStruct((B_TOTAL, D), jnp.float32),
        mesh=mesh,
        scratch_types=[
            pltpu.VMEM((N_CHUNKS, CHUNK), jnp.int32),
            pltpu.VMEM((CHUNK, D), jnp.float32),
            pltpu.SemaphoreType.DMA,
        ],
    )
    def k(doy_hbm, pe_hbm, out_hbm, idx_v, rows_v, gsem):
        wid = lax.axis_index("s") * NC + lax.axis_index("c")
        base = wid * B_PER_W
        pltpu.sync_copy(doy_hbm.at[wid], idx_v)

        def body(g, _):
            pltpu.async_copy(pe_hbm.at[idx_v.at[g]], rows_v, gsem).wait()
            pltpu.sync_copy(rows_v, out_hbm.at[pl.ds(base + g * CHUNK, CHUNK)])
            return ()

        lax.fori_loop(0, N_CHUNKS, body, (), unroll=False)

    return k(doy_r, pe)


def kernel(doy, pe):
    doy_r = doy.reshape(NW, N_CHUNKS, CHUNK).astype(jnp.int32)
    out = _sc_gather(doy_r, pe)
    return out.reshape(B_ROWS, T_COLS, D)


# speedup vs baseline: 4.2150x; 1.0178x over previous
import functools
import jax
import jax.numpy as jnp
from jax import lax
from jax.experimental import pallas as pl
from jax.experimental.pallas import tpu as pltpu
from jax.experimental.pallas import tpu_sc as plsc

D = 128
B_ROWS, T_COLS = 4096, 200
B_TOTAL = B_ROWS * T_COLS
NC, NS = 2, 16
NW = NC * NS
B_PER_W = B_TOTAL // NW
CHUNK = 128
N_CHUNKS = B_PER_W // CHUNK
NB = 4


@jax.jit
def _sc_gather(doy_r, pe):
    mesh = plsc.VectorSubcoreMesh(core_axis_name="c", subcore_axis_name="s")

    @functools.partial(
        pl.kernel,
        out_type=jax.ShapeDtypeStruct((B_TOTAL, D), jnp.float32),
        mesh=mesh,
        scratch_types=[
            pltpu.VMEM((N_CHUNKS, CHUNK), jnp.int32),
            pltpu.VMEM((NB, CHUNK, D), jnp.float32),
            pltpu.SemaphoreType.DMA,
            pltpu.SemaphoreType.DMA,
        ],
    )
    def k(doy_hbm, pe_hbm, out_hbm, idx_v, rows_v, gsem, ssem):
        wid = lax.axis_index("s") * NC + lax.axis_index("c")
        base = wid * B_PER_W
        pltpu.sync_copy(doy_hbm.at[wid], idx_v)

        def start_gather(g, b):
            pltpu.async_copy(pe_hbm.at[idx_v.at[g]], rows_v.at[b], gsem)

        def wait_gather(g, b):
            pltpu.make_async_copy(pe_hbm.at[idx_v.at[g]], rows_v.at[b], gsem).wait()

        def start_scatter(g, b):
            pltpu.async_copy(rows_v.at[b], out_hbm.at[pl.ds(base + g * CHUNK, CHUNK)], ssem)

        def wait_one_scatter():
            pltpu.make_async_copy(rows_v.at[0], out_hbm.at[pl.ds(base, CHUNK)], ssem).wait()

        for b in range(NB):
            start_gather(b, b)
        for g in range(NB - 1):
            wait_gather(g, g)
            start_scatter(g, g)

        def body(o, _):
            for b in range(NB):
                g = (NB - 1) + o * NB + b
                buf = (NB - 1 + b) % NB
                wait_gather(g, buf)
                start_scatter(g, buf)
                wait_one_scatter()
                start_gather(g + 1, (buf + 1) % NB)
            return ()

        lax.fori_loop(0, (N_CHUNKS - NB) // NB, body, (), unroll=False)

        g_last = N_CHUNKS - 1
        wait_gather(g_last, g_last % NB)
        start_scatter(g_last, g_last % NB)
        for _ in range(NB):
            wait_one_scatter()

    return k(doy_r, pe)


def kernel(doy, pe):
    doy_r = doy.reshape(NW, N_CHUNKS, CHUNK).astype(jnp.int32)
    out = _sc_gather(doy_r, pe)
    return out.reshape(B_ROWS, T_COLS, D)


# speedup vs baseline: 14.8924x; 3.5332x over previous
import functools
import jax
import jax.numpy as jnp
from jax import lax
from jax.experimental import pallas as pl
from jax.experimental.pallas import tpu as pltpu
from jax.experimental.pallas import tpu_sc as plsc

D = 128
B_ROWS, T_COLS = 4096, 200
B_TOTAL = B_ROWS * T_COLS
NC, NS = 2, 16
NW = NC * NS
B_PER_W = B_TOTAL // NW
CHUNK = 128
N_CHUNKS = B_PER_W // CHUNK
NB = 4
PE_ROWS = 367


@jax.jit
def _sc_gather(doy_r, pe):
    mesh = plsc.VectorSubcoreMesh(core_axis_name="c", subcore_axis_name="s")

    @functools.partial(
        pl.kernel,
        out_type=jax.ShapeDtypeStruct((B_TOTAL, D), jnp.float32),
        mesh=mesh,
        scratch_types=[
            pltpu.VMEM((N_CHUNKS, CHUNK), jnp.int32),
            pltpu.VMEM((NB, CHUNK, D), jnp.float32),
            pltpu.VMEM_SHARED((PE_ROWS, D), jnp.float32),
            pltpu.SemaphoreType.DMA,
            pltpu.SemaphoreType.DMA,
        ],
    )
    def k(doy_hbm, pe_hbm, out_hbm, idx_v, rows_v, pe_spm, gsem, ssem):
        sid = lax.axis_index("s")
        wid = sid * NC + lax.axis_index("c")
        base = wid * B_PER_W

        @pl.when(sid == 0)
        def _():
            pltpu.sync_copy(pe_hbm, pe_spm)

        pltpu.sync_copy(doy_hbm.at[wid], idx_v)
        plsc.subcore_barrier()

        def start_gather(g, b):
            pltpu.async_copy(pe_spm.at[idx_v.at[g]], rows_v.at[b], gsem)

        def wait_gather(g, b):
            pltpu.make_async_copy(pe_spm.at[idx_v.at[g]], rows_v.at[b], gsem).wait()

        def start_scatter(g, b):
            pltpu.async_copy(rows_v.at[b], out_hbm.at[pl.ds(base + g * CHUNK, CHUNK)], ssem)

        def wait_one_scatter():
            pltpu.make_async_copy(rows_v.at[0], out_hbm.at[pl.ds(base, CHUNK)], ssem).wait()

        for b in range(NB):
            start_gather(b, b)
        for g in range(NB - 1):
            wait_gather(g, g)
            start_scatter(g, g)

        def body(o, _):
            for b in range(NB):
                g = (NB - 1) + o * NB + b
                buf = (NB - 1 + b) % NB
                wait_gather(g, buf)
                start_scatter(g, buf)
                wait_one_scatter()
                start_gather(g + 1, (buf + 1) % NB)
            return ()

        lax.fori_loop(0, (N_CHUNKS - NB) // NB, body, (), unroll=False)

        g_last = N_CHUNKS - 1
        wait_gather(g_last, g_last % NB)
        start_scatter(g_last, g_last % NB)
        for _ in range(NB):
            wait_one_scatter()

    return k(doy_r, pe)


def kernel(doy, pe):
    doy_r = doy.reshape(NW, N_CHUNKS, CHUNK).astype(jnp.int32)
    out = _sc_gather(doy_r, pe)
    return out.reshape(B_ROWS, T_COLS, D)


# speedup vs baseline: 15.5193x; 1.0421x over previous
import functools
import jax
import jax.numpy as jnp
from jax import lax
from jax.experimental import pallas as pl
from jax.experimental.pallas import tpu as pltpu
from jax.experimental.pallas import tpu_sc as plsc

D = 128
B_ROWS, T_COLS = 4096, 200
B_TOTAL = B_ROWS * T_COLS
NC, NS = 2, 16
NW = NC * NS
B_PER_W = B_TOTAL // NW
CHUNK = 128
GPC = 2
OUT_CHUNK = CHUNK * GPC
N_IDX = B_PER_W // CHUNK
G = B_PER_W // OUT_CHUNK
NB = 3
PE_ROWS = 367


@jax.jit
def _sc_gather(doy_r, pe):
    mesh = plsc.VectorSubcoreMesh(core_axis_name="c", subcore_axis_name="s")

    @functools.partial(
        pl.kernel,
        out_type=jax.ShapeDtypeStruct((B_TOTAL, D), jnp.float32),
        mesh=mesh,
        scratch_types=[
            pltpu.VMEM((N_IDX, CHUNK), jnp.int32),
            pltpu.VMEM((NB, OUT_CHUNK, D), jnp.float32),
            pltpu.VMEM_SHARED((PE_ROWS, D), jnp.float32),
            pltpu.SemaphoreType.DMA,
            pltpu.SemaphoreType.DMA,
        ],
    )
    def k(doy_hbm, pe_hbm, out_hbm, idx_v, rows_v, pe_spm, gsem, ssem):
        sid = lax.axis_index("s")
        wid = sid * NC + lax.axis_index("c")
        base = wid * B_PER_W

        @pl.when(sid == 0)
        def _():
            pltpu.sync_copy(pe_hbm, pe_spm)

        pltpu.sync_copy(doy_hbm.at[wid], idx_v)
        plsc.subcore_barrier()

        def start_gathers(g, b):
            for j in range(GPC):
                pltpu.async_copy(pe_spm.at[idx_v.at[GPC * g + j]],
                                 rows_v.at[b].at[pl.ds(j * CHUNK, CHUNK)], gsem)

        def wait_gathers(g, b):
            for j in range(GPC):
                pltpu.make_async_copy(pe_spm.at[idx_v.at[GPC * g + j]],
                                      rows_v.at[b].at[pl.ds(j * CHUNK, CHUNK)], gsem).wait()

        def start_scatter(g, b):
            pltpu.async_copy(rows_v.at[b],
                             out_hbm.at[pl.ds(base + g * OUT_CHUNK, OUT_CHUNK)], ssem)

        def wait_one_scatter():
            pltpu.make_async_copy(rows_v.at[0],
                                  out_hbm.at[pl.ds(base, OUT_CHUNK)], ssem).wait()

        for b in range(NB):
            start_gathers(b, b)
        for g in range(NB - 1):
            wait_gathers(g, g)
            start_scatter(g, g)

        def body(o, _):
            for j in range(NB):
                g = (NB - 1) + o * NB + j
                buf = (NB - 1 + j) % NB
                wait_gathers(g, buf)
                start_scatter(g, buf)
                wait_one_scatter()
                start_gathers(g + 1, (buf + 1) % NB)
            return ()

        n_main = (G - NB) // NB * NB
        lax.fori_loop(0, n_main // NB, body, (), unroll=False)

        for g in range(NB - 1 + n_main, G - 1):
            wait_gathers(g, g % NB)
            start_scatter(g, g % NB)
            wait_one_scatter()
            start_gathers(g + 1, (g + 1) % NB)

        wait_gathers(G - 1, (G - 1) % NB)
        start_scatter(G - 1, (G - 1) % NB)
        for _ in range(NB):
            wait_one_scatter()

    return k(doy_r, pe)


def kernel(doy, pe):
    doy_r = doy.reshape(NW, N_IDX, CHUNK).astype(jnp.int32)
    out = _sc_gather(doy_r, pe)
    return out.reshape(B_ROWS, T_COLS, D)
